# Initial kernel scaffold; baseline (speedup 1.0000x reference)
import jax
import jax.numpy as jnp
from jax.experimental import pallas as pl


def kernel(x, edge_index, patient_ids, treatment, W_gc, b_gc, W_t00, b_t00, W_t10, b_t10, W_t01, b_t01, W_t11, b_t11):
    raise NotImplementedError("write your pallas kernel here")


# speedup vs baseline: 28.5224x; 28.5224x over previous
import functools

import jax
import jax.numpy as jnp
from jax import lax
from jax.experimental import pallas as pl
from jax.experimental.pallas import tpu as pltpu
from jax.experimental.pallas import tpu_sc as plsc

N = 10000
E = 320000
F = 128
NC = 2
NS = 16
NW = NC * NS
EPW = E // NW
CHUNK = 100
NCHUNKS = EPW // CHUNK
ROWS_PER_SUB = 624
B = 5000
BPAD = 5120
BPW = BPAD // NW

_mesh = plsc.VectorSubcoreMesh(
    core_axis_name="c", subcore_axis_name="s", num_cores=NC, num_subcores=NS
)


@functools.partial(
    pl.kernel,
    out_type=jax.ShapeDtypeStruct((NC, N), jnp.float32),
    mesh=_mesh,
    scratch_types=[
        pltpu.VMEM((NCHUNKS, CHUNK), jnp.int32),
        pltpu.VMEM((CHUNK,), jnp.float32),
        pltpu.VMEM_SHARED((N,), jnp.float32),
    ],
)
def _deg_sc(dst_hbm, zeros_hbm, out_hbm, dst_v, ones_v, deg_sh):
    cid = lax.axis_index("c")
    sid = lax.axis_index("s")
    wid = cid * NS + sid
    for j in range(CHUNK // 16):
        ones_v[pl.ds(j * 16, 16)] = jnp.ones((16,), jnp.float32)

    @pl.when(sid == 0)
    def _():
        pltpu.sync_copy(zeros_hbm, deg_sh)

    pltpu.sync_copy(dst_hbm.at[wid], dst_v)
    plsc.subcore_barrier()

    @pl.loop(0, NCHUNKS)
    def _(j):
        pltpu.sync_copy(ones_v, deg_sh.at[dst_v.at[j]], add=True)

    plsc.subcore_barrier()

    @pl.when(sid == 0)
    def _():
        pltpu.sync_copy(deg_sh, out_hbm.at[cid])


def _gmul_body(x_ref, w_ref, cnt_ref, g_ref):
    deg = cnt_ref[0, :] + cnt_ref[1, :] + 1.0
    dinv = lax.rsqrt(deg)[:, None]
    h = jnp.dot(x_ref[...], w_ref[...], preferred_element_type=jnp.float32)
    g_ref[...] = h * dinv


def _gmul(x, w, cnt):
    return pl.pallas_call(
        _gmul_body,
        out_shape=jax.ShapeDtypeStruct((N, F), jnp.float32),
    )(x, w, cnt)


@functools.partial(
    pl.kernel,
    out_type=jax.ShapeDtypeStruct((NC, N, F), jnp.float32),
    mesh=_mesh,
    scratch_types=[
        pltpu.VMEM((NCHUNKS, CHUNK), jnp.int32),
        pltpu.VMEM((NCHUNKS, CHUNK), jnp.int32),
        pltpu.VMEM((CHUNK, F), jnp.float32),
        pltpu.VMEM_SHARED((N, F), jnp.float32),
        pltpu.SemaphoreType.DMA,
    ],
)
def _scat_sc(src_hbm, dst_hbm, g_hbm, zeros_hbm, out_hbm,
             src_v, dst_v, rows_v, acc_sh, sem):
    cid = lax.axis_index("c")
    sid = lax.axis_index("s")
    wid = cid * NS + sid
    r0 = sid * ROWS_PER_SUB
    pltpu.sync_copy(
        zeros_hbm.at[pl.ds(r0, ROWS_PER_SUB)],
        acc_sh.at[pl.ds(r0, ROWS_PER_SUB)],
    )

    @pl.when(sid == NS - 1)
    def _():
        rem = NS * ROWS_PER_SUB
        pltpu.sync_copy(
            zeros_hbm.at[pl.ds(rem, N - rem)],
            acc_sh.at[pl.ds(rem, N - rem)],
        )

    pltpu.sync_copy(src_hbm.at[wid], src_v)
    pltpu.sync_copy(dst_hbm.at[wid], dst_v)
    plsc.subcore_barrier()

    @pl.loop(0, NCHUNKS)
    def _(j):
        pltpu.async_copy(g_hbm.at[src_v.at[j]], rows_v, sem).wait()
        pltpu.sync_copy(rows_v, acc_sh.at[dst_v.at[j]], add=True)

    plsc.subcore_barrier()
    pltpu.sync_copy(
        acc_sh.at[pl.ds(r0, ROWS_PER_SUB)],
        out_hbm.at[cid].at[pl.ds(r0, ROWS_PER_SUB)],
    )

    @pl.when(sid == NS - 1)
    def _():
        rem = NS * ROWS_PER_SUB
        pltpu.sync_copy(
            acc_sh.at[pl.ds(rem, N - rem)],
            out_hbm.at[cid].at[pl.ds(rem, N - rem)],
        )


def _head_body(s_ref, g_ref, cnt_ref, bgc_ref, w00_ref, b00_ref, w10_ref,
               b10_ref, w01_ref, b01_ref, w11_ref, b11_ref, y0_ref, y1_ref):
    deg = cnt_ref[0, :] + cnt_ref[1, :] + 1.0
    dinv = lax.rsqrt(deg)[:, None]
    s = s_ref[0] + s_ref[1] + g_ref[...]
    dist = jnp.maximum(s * dinv + bgc_ref[...][None, :], 0.0)
    y00 = jnp.maximum(
        jnp.dot(dist, w00_ref[...], preferred_element_type=jnp.float32)
        + b00_ref[...][None, :], 0.0)
    y10 = jnp.maximum(
        jnp.dot(dist, w10_ref[...], preferred_element_type=jnp.float32)
        + b10_ref[...][None, :], 0.0)
    z0 = jnp.dot(y00, w01_ref[...], preferred_element_type=jnp.float32)
    z1 = jnp.dot(y10, w11_ref[...], preferred_element_type=jnp.float32)
    y0_ref[...] = jax.nn.sigmoid(z0 + b01_ref[...][None, :])
    y1_ref[...] = jax.nn.sigmoid(z1 + b11_ref[...][None, :])


def _head(s, g, cnt, bgc, w00, b00, w10, b10, w01, b01, w11, b11):
    return pl.pallas_call(
        _head_body,
        out_shape=(
            jax.ShapeDtypeStruct((N, 1), jnp.float32),
            jax.ShapeDtypeStruct((N, 1), jnp.float32),
        ),
    )(s, g, cnt, bgc, w00, b00, w10, b10, w01, b01, w11, b11)


@functools.partial(
    pl.kernel,
    out_type=(
        jax.ShapeDtypeStruct((BPAD,), jnp.float32),
        jax.ShapeDtypeStruct((BPAD,), jnp.float32),
        jax.ShapeDtypeStruct((BPAD,), jnp.float32),
    ),
    mesh=_mesh,
    scratch_types=[
        pltpu.VMEM((BPW,), jnp.int32),
        pltpu.VMEM((BPW,), jnp.int32),
        pltpu.VMEM((BPW,), jnp.float32),
        pltpu.VMEM((BPW,), jnp.float32),
        pltpu.VMEM((BPW,), jnp.float32),
        pltpu.SemaphoreType.DMA,
    ],
)
def _pick_sc(y0_hbm, y1_hbm, pid_hbm, t_hbm, y_out, y1_out, y0_out,
             pid_v, t_v, g0_v, g1_v, oy_v, sem):
    cid = lax.axis_index("c")
    sid = lax.axis_index("s")
    wid = cid * NS + sid
    base = wid * BPW
    pltpu.sync_copy(pid_hbm.at[pl.ds(base, BPW)], pid_v)
    pltpu.sync_copy(t_hbm.at[pl.ds(base, BPW)], t_v)
    pltpu.async_copy(y0_hbm.at[pid_v], g0_v, sem).wait()
    pltpu.async_copy(y1_hbm.at[pid_v], g1_v, sem).wait()
    for i in range(BPW // 16):
        sl = pl.ds(i * 16, 16)
        oy_v[sl] = jnp.where(t_v[sl] > 0, g1_v[sl], g0_v[sl])
    pltpu.sync_copy(oy_v, y_out.at[pl.ds(base, BPW)])
    pltpu.sync_copy(g1_v, y1_out.at[pl.ds(base, BPW)])
    pltpu.sync_copy(g0_v, y0_out.at[pl.ds(base, BPW)])


def kernel(x, edge_index, patient_ids, treatment, W_gc, b_gc, W_t00, b_t00,
           W_t10, b_t10, W_t01, b_t01, W_t11, b_t11):
    src = edge_index[0].astype(jnp.int32).reshape(NW, NCHUNKS, CHUNK)
    dst = edge_index[1].astype(jnp.int32).reshape(NW, NCHUNKS, CHUNK)
    zeros_n = jnp.zeros((N,), jnp.float32)
    zeros_nf = jnp.zeros((N, F), jnp.float32)

    cnt = _deg_sc(dst, zeros_n)
    g = _gmul(x, W_gc, cnt)
    s = _scat_sc(src, dst, g, zeros_nf)
    y0, y1 = _head(s, g, cnt, b_gc, W_t00, b_t00, W_t10, b_t10,
                   W_t01, b_t01, W_t11, b_t11)
    y0 = y0.reshape(N)
    y1 = y1.reshape(N)

    pad_i = jnp.zeros((BPAD - B,), jnp.int32)
    pid = jnp.concatenate([patient_ids.astype(jnp.int32), pad_i])
    tre = jnp.concatenate([treatment.astype(jnp.int32), pad_i])
    y, y1p, y0p = _pick_sc(y0, y1, pid, tre)
    return y[:B], y1p[:B], y0p[:B]


# speedup vs baseline: 34.2571x; 1.2011x over previous
import functools

import jax
import jax.numpy as jnp
from jax import lax
from jax.experimental import pallas as pl
from jax.experimental.pallas import tpu as pltpu
from jax.experimental.pallas import tpu_sc as plsc

N = 10000
E = 320000
F = 128
NC = 2
NS = 16
NW = NC * NS
EPW = E // NW
CHUNK = 100
NH = 2
NCH = EPW // (NH * CHUNK)
ROWS_PER_SUB = 624
B = 5000
BPAD = 5120
BPW = BPAD // NW

_mesh = plsc.VectorSubcoreMesh(
    core_axis_name="c", subcore_axis_name="s", num_cores=NC, num_subcores=NS
)


@functools.partial(
    pl.kernel,
    out_type=jax.ShapeDtypeStruct((NC, N), jnp.float32),
    mesh=_mesh,
    scratch_types=[
        pltpu.VMEM((NCH, CHUNK), jnp.int32),
        pltpu.VMEM((CHUNK,), jnp.float32),
        pltpu.VMEM_SHARED((N,), jnp.float32),
    ],
)
def _deg_sc(dst_hbm, zeros_hbm, out_hbm, dst_v, ones_v, deg_sh):
    cid = lax.axis_index("c")
    sid = lax.axis_index("s")
    wid = cid * NS + sid
    for j in range(CHUNK // 16):
        ones_v[pl.ds(j * 16, 16)] = jnp.ones((16,), jnp.float32)

    @pl.when(sid == 0)
    def _():
        pltpu.sync_copy(zeros_hbm, deg_sh)

    plsc.subcore_barrier()
    for h in range(NH):
        pltpu.sync_copy(dst_hbm.at[wid, h], dst_v)

        @pl.loop(0, NCH)
        def _(j):
            pltpu.sync_copy(ones_v, deg_sh.at[dst_v.at[j]], add=True)

    plsc.subcore_barrier()

    @pl.when(sid == 0)
    def _():
        pltpu.sync_copy(deg_sh, out_hbm.at[cid])


def _gmul_body(x_ref, w_ref, cnt_ref, g_ref):
    deg = cnt_ref[0, :] + cnt_ref[1, :] + 1.0
    dinv = lax.rsqrt(deg)[:, None]
    h = jnp.dot(x_ref[...], w_ref[...], preferred_element_type=jnp.float32)
    g_ref[...] = h * dinv


def _gmul(x, w, cnt):
    return pl.pallas_call(
        _gmul_body,
        out_shape=jax.ShapeDtypeStruct((N, F), jnp.float32),
    )(x, w, cnt)


@functools.partial(
    pl.kernel,
    out_type=jax.ShapeDtypeStruct((NC, N, F), jnp.float32),
    mesh=_mesh,
    scratch_types=[
        pltpu.VMEM((NCH, CHUNK), jnp.int32),
        pltpu.VMEM((NCH, CHUNK), jnp.int32),
        pltpu.VMEM((CHUNK, F), jnp.float32),
        pltpu.VMEM((CHUNK, F), jnp.float32),
        pltpu.VMEM_SHARED((N, F), jnp.float32),
        pltpu.SemaphoreType.DMA,
        pltpu.SemaphoreType.DMA,
    ],
)
def _scat_sc(src_hbm, dst_hbm, g_hbm, zeros_hbm, out_hbm,
             src_v, dst_v, rows0_v, rows1_v, acc_sh, sem0, sem1):
    cid = lax.axis_index("c")
    sid = lax.axis_index("s")
    wid = cid * NS + sid
    r0 = sid * ROWS_PER_SUB
    pltpu.sync_copy(
        zeros_hbm.at[pl.ds(r0, ROWS_PER_SUB)],
        acc_sh.at[pl.ds(r0, ROWS_PER_SUB)],
    )

    @pl.when(sid == NS - 1)
    def _():
        rem = NS * ROWS_PER_SUB
        pltpu.sync_copy(
            zeros_hbm.at[pl.ds(rem, N - rem)],
            acc_sh.at[pl.ds(rem, N - rem)],
        )

    plsc.subcore_barrier()

    def gstart(j, buf, sem):
        pltpu.make_async_copy(g_hbm.at[src_v.at[j]], buf, sem).start()

    def gwait(buf, sem):
        pltpu.make_async_copy(g_hbm.at[src_v.at[0]], buf, sem).wait()

    def scat(buf, j):
        pltpu.sync_copy(buf, acc_sh.at[dst_v.at[j]], add=True)

    for h in range(NH):
        pltpu.sync_copy(src_hbm.at[wid, h], src_v)
        pltpu.sync_copy(dst_hbm.at[wid, h], dst_v)
        gstart(0, rows0_v, sem0)

        @pl.loop(0, NCH // 2 - 1)
        def _(p):
            j0 = 2 * p
            gwait(rows0_v, sem0)
            gstart(j0 + 1, rows1_v, sem1)
            scat(rows0_v, j0)
            gwait(rows1_v, sem1)
            gstart(j0 + 2, rows0_v, sem0)
            scat(rows1_v, j0 + 1)

        gwait(rows0_v, sem0)
        gstart(NCH - 1, rows1_v, sem1)
        scat(rows0_v, NCH - 2)
        gwait(rows1_v, sem1)
        scat(rows1_v, NCH - 1)

    plsc.subcore_barrier()
    pltpu.sync_copy(
        acc_sh.at[pl.ds(r0, ROWS_PER_SUB)],
        out_hbm.at[cid].at[pl.ds(r0, ROWS_PER_SUB)],
    )

    @pl.when(sid == NS - 1)
    def _():
        rem = NS * ROWS_PER_SUB
        pltpu.sync_copy(
            acc_sh.at[pl.ds(rem, N - rem)],
            out_hbm.at[cid].at[pl.ds(rem, N - rem)],
        )


def _head_body(s_ref, g_ref, cnt_ref, bgc_ref, w00_ref, b00_ref, w10_ref,
               b10_ref, w01_ref, b01_ref, w11_ref, b11_ref, y0_ref, y1_ref):
    deg = cnt_ref[0, :] + cnt_ref[1, :] + 1.0
    dinv = lax.rsqrt(deg)[:, None]
    s = s_ref[0] + s_ref[1] + g_ref[...]
    dist = jnp.maximum(s * dinv + bgc_ref[...][None, :], 0.0)
    y00 = jnp.maximum(
        jnp.dot(dist, w00_ref[...], preferred_element_type=jnp.float32)
        + b00_ref[...][None, :], 0.0)
    y10 = jnp.maximum(
        jnp.dot(dist, w10_ref[...], preferred_element_type=jnp.float32)
        + b10_ref[...][None, :], 0.0)
    z0 = jnp.dot(y00, w01_ref[...], preferred_element_type=jnp.float32)
    z1 = jnp.dot(y10, w11_ref[...], preferred_element_type=jnp.float32)
    y0_ref[...] = jax.nn.sigmoid(z0 + b01_ref[...][None, :])
    y1_ref[...] = jax.nn.sigmoid(z1 + b11_ref[...][None, :])


def _head(s, g, cnt, bgc, w00, b00, w10, b10, w01, b01, w11, b11):
    return pl.pallas_call(
        _head_body,
        out_shape=(
            jax.ShapeDtypeStruct((N, 1), jnp.float32),
            jax.ShapeDtypeStruct((N, 1), jnp.float32),
        ),
    )(s, g, cnt, bgc, w00, b00, w10, b10, w01, b01, w11, b11)


@functools.partial(
    pl.kernel,
    out_type=(
        jax.ShapeDtypeStruct((BPAD,), jnp.float32),
        jax.ShapeDtypeStruct((BPAD,), jnp.float32),
        jax.ShapeDtypeStruct((BPAD,), jnp.float32),
    ),
    mesh=_mesh,
    scratch_types=[
        pltpu.VMEM((BPW,), jnp.int32),
        pltpu.VMEM((BPW,), jnp.int32),
        pltpu.VMEM((BPW,), jnp.float32),
        pltpu.VMEM((BPW,), jnp.float32),
        pltpu.VMEM((BPW,), jnp.float32),
        pltpu.SemaphoreType.DMA,
    ],
)
def _pick_sc(y0_hbm, y1_hbm, pid_hbm, t_hbm, y_out, y1_out, y0_out,
             pid_v, t_v, g0_v, g1_v, oy_v, sem):
    cid = lax.axis_index("c")
    sid = lax.axis_index("s")
    wid = cid * NS + sid
    base = wid * BPW
    pltpu.sync_copy(pid_hbm.at[pl.ds(base, BPW)], pid_v)
    pltpu.sync_copy(t_hbm.at[pl.ds(base, BPW)], t_v)
    pltpu.async_copy(y0_hbm.at[pid_v], g0_v, sem).wait()
    pltpu.async_copy(y1_hbm.at[pid_v], g1_v, sem).wait()
    for i in range(BPW // 16):
        sl = pl.ds(i * 16, 16)
        oy_v[sl] = jnp.where(t_v[sl] > 0, g1_v[sl], g0_v[sl])
    pltpu.sync_copy(oy_v, y_out.at[pl.ds(base, BPW)])
    pltpu.sync_copy(g1_v, y1_out.at[pl.ds(base, BPW)])
    pltpu.sync_copy(g0_v, y0_out.at[pl.ds(base, BPW)])


def kernel(x, edge_index, patient_ids, treatment, W_gc, b_gc, W_t00, b_t00,
           W_t10, b_t10, W_t01, b_t01, W_t11, b_t11):
    src = edge_index[0].astype(jnp.int32).reshape(NW, NH, NCH, CHUNK)
    dst = edge_index[1].astype(jnp.int32).reshape(NW, NH, NCH, CHUNK)
    zeros_n = jnp.zeros((N,), jnp.float32)
    zeros_nf = jnp.zeros((N, F), jnp.float32)

    cnt = _deg_sc(dst, zeros_n)
    g = _gmul(x, W_gc, cnt)
    s = _scat_sc(src, dst, g, zeros_nf)
    y0, y1 = _head(s, g, cnt, b_gc, W_t00, b_t00, W_t10, b_t10,
                   W_t01, b_t01, W_t11, b_t11)
    y0 = y0.reshape(N)
    y1 = y1.reshape(N)

    pad_i = jnp.zeros((BPAD - B,), jnp.int32)
    pid = jnp.concatenate([patient_ids.astype(jnp.int32), pad_i])
    tre = jnp.concatenate([treatment.astype(jnp.int32), pad_i])
    y, y1p, y0p = _pick_sc(y0, y1, pid, tre)
    return y[:B], y1p[:B], y0p[:B]
